# Initial kernel scaffold; baseline (speedup 1.0000x reference)
import jax
import jax.numpy as jnp
from jax.experimental import pallas as pl


def kernel(x, edge_index, edge_attr, W1, as1, ad1, We1, ae1, b1, W2, as2, ad2, We2, ae2, b2, W3, as3, ad3, We3, ae3, b3):
    raise NotImplementedError("write your pallas kernel here")


# speedup vs baseline: 13.5150x; 13.5150x over previous
import functools

import jax
import jax.numpy as jnp
from jax import lax
from jax.experimental import pallas as pl
from jax.experimental.pallas import tpu as pltpu
from jax.experimental.pallas import tpu_sc as plsc

N = 10000
E = 320000
D = 128
DE = 16

NC = 2
NS = 16
NW = NC * NS

EPT = 10240
EPAD = EPT * NW
CH = 128
NCHUNK = EPT // CH
NROWPAD = 10240
RPT = NROWPAD // NS
NEG = -1.0e9

BN = 2000
BE = 6400


def _aedge_body(ea_ref, We1, ae1, We2, ae2, We3, ae3, out_ref, mean_ref):
    i = pl.program_id(0)
    wv1 = jnp.dot(We1[...], ae1[...])
    wv2 = jnp.dot(We2[...], ae2[...])
    wv3 = jnp.dot(We3[...], ae3[...])
    wv = jnp.concatenate([wv1, wv2, wv3], axis=1)
    blk = lax.dot_general(wv, ea_ref[...],
                          (((0,), (1,)), ((), ())),
                          preferred_element_type=jnp.float32)
    out_ref[...] = blk
    part = jnp.sum(blk, axis=1, keepdims=True)

    @pl.when(i == 0)
    def _():
        mean_ref[...] = jnp.zeros_like(mean_ref)

    mean_ref[...] += jnp.broadcast_to(part, mean_ref.shape)

    @pl.when(i == (E // BE) - 1)
    def _():
        mean_ref[...] = mean_ref[...] * (1.0 / E)


def _aedge_call(edge_attr, We1, ae1, We2, ae2, We3, ae3):
    full = lambda s: pl.BlockSpec(s, lambda i: (0, 0))
    return pl.pallas_call(
        _aedge_body,
        grid=(E // BE,),
        in_specs=[
            pl.BlockSpec((BE, DE), lambda i: (i, 0)),
            full((DE, D)), full((D, 1)),
            full((DE, D)), full((D, 1)),
            full((DE, D)), full((D, 1)),
        ],
        out_specs=[
            pl.BlockSpec((3, BE), lambda i: (0, i)),
            pl.BlockSpec((3, 128), lambda i: (0, 0)),
        ],
        out_shape=[
            jax.ShapeDtypeStruct((3, E), jnp.float32),
            jax.ShapeDtypeStruct((3, 128), jnp.float32),
        ],
    )(edge_attr, We1, ae1, We2, ae2, We3, ae3)


def _pre_body(x_ref, W_ref, as_ref, ad_ref, h_ref, av_ref):
    h = jnp.dot(x_ref[...], W_ref[...], preferred_element_type=jnp.float32)
    h_ref[...] = h
    asv = jnp.dot(h, as_ref[...], preferred_element_type=jnp.float32)
    adv = jnp.dot(h, ad_ref[...], preferred_element_type=jnp.float32)
    av_ref[...] = jnp.concatenate(
        [asv, adv, jnp.zeros((asv.shape[0], 6), jnp.float32)], axis=1)


def _pre_call(x, W, a_s, a_d):
    full = lambda s: pl.BlockSpec(s, lambda i: (0, 0))
    return pl.pallas_call(
        _pre_body,
        grid=(N // BN,),
        in_specs=[
            pl.BlockSpec((BN, D), lambda i: (i, 0)),
            full((D, D)), full((D, 1)), full((D, 1)),
        ],
        out_specs=[
            pl.BlockSpec((BN, D), lambda i: (i, 0)),
            pl.BlockSpec((BN, 8), lambda i: (i, 0)),
        ],
        out_shape=[
            jax.ShapeDtypeStruct((N, D), jnp.float32),
            jax.ShapeDtypeStruct((N, 8), jnp.float32),
        ],
    )(x, W, a_s, a_d)


def _epi_body(has_prev, has_next, is_final, *refs):
    refs = list(refs)
    u0, u1, d0, d1, h_ref, av_ref, am_ref, b_ref = refs[:8]
    refs = refs[8:]
    if has_prev:
        prev_ref = refs.pop(0)
    if has_next:
        Wn_ref, asn_ref, adn_ref = refs[:3]
        refs = refs[3:]
    ssum_ref = refs.pop(0)
    if has_next:
        hn_ref, avn_ref = refs[:2]
        refs = refs[2:]
    if is_final:
        ge_ref = refs.pop(0)

    av = av_ref[...]
    z = av[:, 0:1] + av[:, 1:2] + am_ref[0, 0]
    z = jnp.maximum(z, 0.2 * z)
    exl = jnp.exp(z)
    h = h_ref[...]
    U = u0[...] + u1[...] + exl * h
    den = d0[...] + d1[...] + exl
    o = U / (den + 1e-16) + b_ref[...]
    xl = jnp.where(o > 0, o, jnp.exp(jnp.minimum(o, 0.0)) - 1.0)
    s = prev_ref[...] + xl if has_prev else xl
    ssum_ref[...] = s
    if has_next:
        hn = jnp.dot(xl, Wn_ref[...], preferred_element_type=jnp.float32)
        hn_ref[...] = hn
        asv = jnp.dot(hn, asn_ref[...], preferred_element_type=jnp.float32)
        adv = jnp.dot(hn, adn_ref[...], preferred_element_type=jnp.float32)
        avn_ref[...] = jnp.concatenate(
            [asv, adv, jnp.zeros((asv.shape[0], 6), jnp.float32)], axis=1)
    if is_final:
        i = pl.program_id(0)

        @pl.when(i == 0)
        def _():
            ge_ref[...] = jnp.zeros_like(ge_ref)

        ge_ref[...] += jnp.sum(s, axis=0, keepdims=True)

        @pl.when(i == (N // BN) - 1)
        def _():
            ge_ref[...] = ge_ref[...] * (1.0 / N)


def _epi_call(u0, u1, d0, d1, h, av, amean, b, prev=None, nxt=None,
              is_final=False):
    has_prev = prev is not None
    has_next = nxt is not None
    full = lambda s: pl.BlockSpec(s, lambda i: (0, 0))
    nb = lambda c: pl.BlockSpec((BN, c), lambda i: (i, 0))
    in_specs = [nb(D), nb(D), nb(1), nb(1), nb(D), nb(8),
                full((1, 1)), full((1, D))]
    args = [u0, u1, d0, d1, h, av, amean, b]
    if has_prev:
        in_specs.append(nb(D))
        args.append(prev)
    if has_next:
        in_specs += [full((D, D)), full((D, 1)), full((D, 1))]
        args += list(nxt)
    out_specs = [nb(D)]
    out_shape = [jax.ShapeDtypeStruct((N, D), jnp.float32)]
    if has_next:
        out_specs += [nb(D), nb(8)]
        out_shape += [jax.ShapeDtypeStruct((N, D), jnp.float32),
                      jax.ShapeDtypeStruct((N, 8), jnp.float32)]
    if is_final:
        out_specs.append(pl.BlockSpec((1, D), lambda i: (0, 0)))
        out_shape.append(jax.ShapeDtypeStruct((1, D), jnp.float32))
    body = functools.partial(_epi_body, has_prev, has_next, is_final)
    return pl.pallas_call(
        body,
        grid=(N // BN,),
        in_specs=in_specs,
        out_specs=out_specs,
        out_shape=out_shape,
    )(*args)


def _sc_edge_body(src_hbm, dst_hbm, ae_hbm, asrc_hbm, adst_hbm, h_hbm,
                  u_out, den_out,
                  asrc_t, adst_t, src_b, dst_b, ae_b, ex_b, rows_b, zrow_b,
                  zden_b, u_sh, den_sh, sem):
    cid = lax.axis_index("c")
    sid = lax.axis_index("s")
    w = cid * NS + sid

    zero16 = jnp.zeros((16,), jnp.float32)
    for i in range(16):
        for k in range(D // 16):
            zrow_b[i, pl.ds(k * 16, 16)] = zero16
    for k in range(RPT // 16):
        zden_b[pl.ds(k * 16, 16)] = zero16

    def zero_body(r, _):
        pltpu.sync_copy(zrow_b, u_sh.at[pl.ds(sid * RPT + r * 16, 16)])
        return 0

    lax.fori_loop(0, RPT // 16, zero_body, 0)
    pltpu.sync_copy(zden_b, den_sh.at[pl.ds(sid * RPT, RPT)])

    pltpu.sync_copy(asrc_hbm, asrc_t)
    pltpu.sync_copy(adst_hbm, adst_t)

    plsc.subcore_barrier()

    def chunk_body(g, _):
        base = w * EPT + g * CH
        pltpu.sync_copy(src_hbm.at[pl.ds(base, CH)], src_b)
        pltpu.sync_copy(dst_hbm.at[pl.ds(base, CH)], dst_b)
        pltpu.sync_copy(ae_hbm.at[pl.ds(base, CH)], ae_b)
        pltpu.async_copy(h_hbm.at[src_b], rows_b, sem).wait()
        for k in range(CH // 16):
            s16 = src_b[pl.ds(k * 16, 16)]
            d16 = dst_b[pl.ds(k * 16, 16)]
            a = (plsc.load_gather(asrc_t, [s16])
                 + plsc.load_gather(adst_t, [d16])
                 + ae_b[pl.ds(k * 16, 16)])
            a = jnp.maximum(a, 0.2 * a)
            ex_b[pl.ds(k * 16, 16)] = jnp.exp(a)

        def scale_body(j, _):
            idx16 = jnp.broadcast_to(j, (16,)).astype(jnp.int32)
            sc = plsc.load_gather(ex_b, [idx16])
            for k in range(D // 16):
                rows_b[j, pl.ds(k * 16, 16)] = rows_b[j, pl.ds(k * 16, 16)] * sc
            return 0

        lax.fori_loop(0, CH, scale_body, 0)
        pltpu.sync_copy(rows_b, u_sh.at[dst_b], add=True)
        pltpu.sync_copy(ex_b, den_sh.at[dst_b], add=True)
        return 0

    lax.fori_loop(0, NCHUNK, chunk_body, 0)

    plsc.subcore_barrier()

    pltpu.sync_copy(u_sh.at[pl.ds(sid * RPT, RPT)],
                    u_out.at[pl.ds(cid * NROWPAD + sid * RPT, RPT)])
    pltpu.sync_copy(den_sh.at[pl.ds(sid * RPT, RPT)],
                    den_out.at[pl.ds(cid * NROWPAD + sid * RPT, RPT)])


@functools.cache
def _sc_edge_kernel():
    return pl.kernel(
        _sc_edge_body,
        out_type=(jax.ShapeDtypeStruct((2 * NROWPAD, D), jnp.float32),
                  jax.ShapeDtypeStruct((2 * NROWPAD,), jnp.float32)),
        mesh=plsc.VectorSubcoreMesh(core_axis_name="c", subcore_axis_name="s",
                                    num_cores=NC, num_subcores=NS),
        compiler_params=pltpu.CompilerParams(needs_layout_passes=False),
        scratch_types=[
            pltpu.VMEM((N,), jnp.float32),
            pltpu.VMEM((N,), jnp.float32),
            pltpu.VMEM((CH,), jnp.int32),
            pltpu.VMEM((CH,), jnp.int32),
            pltpu.VMEM((CH,), jnp.float32),
            pltpu.VMEM((CH,), jnp.float32),
            pltpu.VMEM((CH, D), jnp.float32),
            pltpu.VMEM((16, D), jnp.float32),
            pltpu.VMEM((RPT,), jnp.float32),
            pltpu.VMEM_SHARED((NROWPAD, D), jnp.float32),
            pltpu.VMEM_SHARED((NROWPAD,), jnp.float32),
            pltpu.SemaphoreType.DMA,
        ],
    )


def _sc_edge(*args):
    return _sc_edge_kernel()(*args)


def kernel(x, edge_index, edge_attr,
           W1, as1, ad1, We1, ae1, b1,
           W2, as2, ad2, We2, ae2, b2,
           W3, as3, ad3, We3, ae3, b3):
    pad = EPAD - E
    src = jnp.concatenate([edge_index[0], jnp.zeros((pad,), jnp.int32)])
    dst = jnp.concatenate([edge_index[1], jnp.zeros((pad,), jnp.int32)])

    col = lambda v: v.reshape(D, 1)
    aedge3, amean3 = _aedge_call(edge_attr, We1, col(ae1), We2, col(ae2),
                                 We3, col(ae3))
    aepad = jnp.full((3, pad), NEG, jnp.float32)
    aedge3 = jnp.concatenate([aedge3, aepad], axis=1)

    def run_layer(h, av, aedge, amean, b, prev, nxt, is_final):
        asrc = av[:, 0]
        adst = av[:, 1]
        u_all, den_all = _sc_edge(src, dst, aedge, asrc, adst, h)
        u0 = u_all[:N]
        u1 = u_all[NROWPAD:NROWPAD + N]
        d0 = den_all[:N].reshape(N, 1)
        d1 = den_all[NROWPAD:NROWPAD + N].reshape(N, 1)
        return _epi_call(u0, u1, d0, d1, h, av, amean, b.reshape(1, D),
                         prev=prev, nxt=nxt, is_final=is_final)

    h1, av1 = _pre_call(x, W1, col(as1), col(ad1))
    s1, h2, av2 = run_layer(h1, av1, aedge3[0], amean3[0:1, 0:1], b1,
                            None, (W2, col(as2), col(ad2)), False)
    s2, h3, av3 = run_layer(h2, av2, aedge3[1], amean3[1:2, 0:1], b2,
                            s1, (W3, col(as3), col(ad3)), False)
    node_emb, ge = run_layer(h3, av3, aedge3[2], amean3[2:3, 0:1], b3,
                             s2, None, True)
    return node_emb, ge


# speedup vs baseline: 16.6729x; 1.2337x over previous
import functools

import jax
import jax.numpy as jnp
from jax import lax
from jax.experimental import pallas as pl
from jax.experimental.pallas import tpu as pltpu
from jax.experimental.pallas import tpu_sc as plsc

N = 10000
E = 320000
D = 128
DE = 16

NC = 2
NS = 16
NW = NC * NS

EPT = 10240
EPAD = EPT * NW
CH = 80
NCHUNK = EPT // CH
NROWPAD = 10240
RPT = NROWPAD // NS
NEG = -1.0e9

BN = 2000
BE = 6400


def _aedge_body(ea_ref, We1, ae1, We2, ae2, We3, ae3, out_ref, mean_ref):
    i = pl.program_id(0)
    wv1 = jnp.dot(We1[...], ae1[...])
    wv2 = jnp.dot(We2[...], ae2[...])
    wv3 = jnp.dot(We3[...], ae3[...])
    wv = jnp.concatenate([wv1, wv2, wv3], axis=1)
    blk = lax.dot_general(wv, ea_ref[...],
                          (((0,), (1,)), ((), ())),
                          preferred_element_type=jnp.float32)
    out_ref[...] = blk
    part = jnp.sum(blk, axis=1, keepdims=True)

    @pl.when(i == 0)
    def _():
        mean_ref[...] = jnp.zeros_like(mean_ref)

    mean_ref[...] += jnp.broadcast_to(part, mean_ref.shape)

    @pl.when(i == (E // BE) - 1)
    def _():
        mean_ref[...] = mean_ref[...] * (1.0 / E)


def _aedge_call(edge_attr, We1, ae1, We2, ae2, We3, ae3):
    full = lambda s: pl.BlockSpec(s, lambda i: (0, 0))
    return pl.pallas_call(
        _aedge_body,
        grid=(E // BE,),
        in_specs=[
            pl.BlockSpec((BE, DE), lambda i: (i, 0)),
            full((DE, D)), full((D, 1)),
            full((DE, D)), full((D, 1)),
            full((DE, D)), full((D, 1)),
        ],
        out_specs=[
            pl.BlockSpec((3, BE), lambda i: (0, i)),
            pl.BlockSpec((3, 128), lambda i: (0, 0)),
        ],
        out_shape=[
            jax.ShapeDtypeStruct((3, E), jnp.float32),
            jax.ShapeDtypeStruct((3, 128), jnp.float32),
        ],
    )(edge_attr, We1, ae1, We2, ae2, We3, ae3)


def _pre_body(x_ref, W_ref, as_ref, ad_ref, h_ref, av_ref):
    h = jnp.dot(x_ref[...], W_ref[...], preferred_element_type=jnp.float32)
    h_ref[...] = h
    asv = jnp.dot(h, as_ref[...], preferred_element_type=jnp.float32)
    adv = jnp.dot(h, ad_ref[...], preferred_element_type=jnp.float32)
    av_ref[...] = jnp.concatenate(
        [asv, adv, jnp.zeros((asv.shape[0], 6), jnp.float32)], axis=1)


def _pre_call(x, W, a_s, a_d):
    full = lambda s: pl.BlockSpec(s, lambda i: (0, 0))
    return pl.pallas_call(
        _pre_body,
        grid=(N // BN,),
        in_specs=[
            pl.BlockSpec((BN, D), lambda i: (i, 0)),
            full((D, D)), full((D, 1)), full((D, 1)),
        ],
        out_specs=[
            pl.BlockSpec((BN, D), lambda i: (i, 0)),
            pl.BlockSpec((BN, 8), lambda i: (i, 0)),
        ],
        out_shape=[
            jax.ShapeDtypeStruct((N, D), jnp.float32),
            jax.ShapeDtypeStruct((N, 8), jnp.float32),
        ],
    )(x, W, a_s, a_d)


def _epi_body(has_prev, has_next, is_final, *refs):
    refs = list(refs)
    u0, u1, d0, d1, h_ref, av_ref, am_ref, b_ref = refs[:8]
    refs = refs[8:]
    if has_prev:
        prev_ref = refs.pop(0)
    if has_next:
        Wn_ref, asn_ref, adn_ref = refs[:3]
        refs = refs[3:]
    ssum_ref = refs.pop(0)
    if has_next:
        hn_ref, avn_ref = refs[:2]
        refs = refs[2:]
    if is_final:
        ge_ref = refs.pop(0)

    av = av_ref[...]
    z = av[:, 0:1] + av[:, 1:2] + am_ref[0, 0]
    z = jnp.maximum(z, 0.2 * z)
    exl = jnp.exp(z)
    h = h_ref[...]
    U = u0[...] + u1[...] + exl * h
    den = d0[...] + d1[...] + exl
    o = U / (den + 1e-16) + b_ref[...]
    xl = jnp.where(o > 0, o, jnp.exp(jnp.minimum(o, 0.0)) - 1.0)
    s = prev_ref[...] + xl if has_prev else xl
    ssum_ref[...] = s
    if has_next:
        hn = jnp.dot(xl, Wn_ref[...], preferred_element_type=jnp.float32)
        hn_ref[...] = hn
        asv = jnp.dot(hn, asn_ref[...], preferred_element_type=jnp.float32)
        adv = jnp.dot(hn, adn_ref[...], preferred_element_type=jnp.float32)
        avn_ref[...] = jnp.concatenate(
            [asv, adv, jnp.zeros((asv.shape[0], 6), jnp.float32)], axis=1)
    if is_final:
        i = pl.program_id(0)

        @pl.when(i == 0)
        def _():
            ge_ref[...] = jnp.zeros_like(ge_ref)

        ge_ref[...] += jnp.sum(s, axis=0, keepdims=True)

        @pl.when(i == (N // BN) - 1)
        def _():
            ge_ref[...] = ge_ref[...] * (1.0 / N)


def _epi_call(u0, u1, d0, d1, h, av, amean, b, prev=None, nxt=None,
              is_final=False):
    has_prev = prev is not None
    has_next = nxt is not None
    full = lambda s: pl.BlockSpec(s, lambda i: (0, 0))
    nb = lambda c: pl.BlockSpec((BN, c), lambda i: (i, 0))
    in_specs = [nb(D), nb(D), nb(1), nb(1), nb(D), nb(8),
                full((1, 1)), full((1, D))]
    args = [u0, u1, d0, d1, h, av, amean, b]
    if has_prev:
        in_specs.append(nb(D))
        args.append(prev)
    if has_next:
        in_specs += [full((D, D)), full((D, 1)), full((D, 1))]
        args += list(nxt)
    out_specs = [nb(D)]
    out_shape = [jax.ShapeDtypeStruct((N, D), jnp.float32)]
    if has_next:
        out_specs += [nb(D), nb(8)]
        out_shape += [jax.ShapeDtypeStruct((N, D), jnp.float32),
                      jax.ShapeDtypeStruct((N, 8), jnp.float32)]
    if is_final:
        out_specs.append(pl.BlockSpec((1, D), lambda i: (0, 0)))
        out_shape.append(jax.ShapeDtypeStruct((1, D), jnp.float32))
    body = functools.partial(_epi_body, has_prev, has_next, is_final)
    return pl.pallas_call(
        body,
        grid=(N // BN,),
        in_specs=in_specs,
        out_specs=out_specs,
        out_shape=out_shape,
    )(*args)


def _sc_edge_body(src_hbm, dst_hbm, ae_hbm, asrc_hbm, adst_hbm, h_hbm,
                  u_out, den_out,
                  asrc_t, adst_t,
                  sb0, sb1, sb2, sb3, db0, db1, db2, db3, ab0, ab1, ab2, ab3,
                  ex0, ex1, rows0, rows1, u_sh, den_sh,
                  *sems):
    cid = lax.axis_index("c")
    sid = lax.axis_index("s")
    w = cid * NS + sid
    srcs = [sb0, sb1, sb2, sb3]
    dsts = [db0, db1, db2, db3]
    aes = [ab0, ab1, ab2, ab3]
    exs = [ex0, ex1]
    rows = [rows0, rows1]
    lin_sems = sems[0:4]
    gat_sems = sems[4:6]
    rsc_sems = sems[6:8]
    dsc_sems = sems[8:10]

    zero16 = jnp.zeros((16,), jnp.float32)

    def zfill_body(r, _):
        for k in range(D // 16):
            rows0[r, pl.ds(k * 16, 16)] = zero16
        return 0

    lax.fori_loop(0, CH, zfill_body, 0)
    for k in range(CH // 16):
        ex0[pl.ds(k * 16, 16)] = zero16

    def zero_body(r, _):
        pltpu.sync_copy(rows0, u_sh.at[pl.ds(sid * RPT + r * CH, CH)])
        pltpu.sync_copy(ex0, den_sh.at[pl.ds(sid * RPT + r * CH, CH)])
        return 0

    lax.fori_loop(0, RPT // CH, zero_body, 0)

    pltpu.sync_copy(asrc_hbm, asrc_t)
    pltpu.sync_copy(adst_hbm, adst_t)

    plsc.subcore_barrier()

    def start_lin(g, s):
        base = w * EPT + g * CH
        pltpu.async_copy(src_hbm.at[pl.ds(base, CH)], srcs[s], lin_sems[s])
        pltpu.async_copy(dst_hbm.at[pl.ds(base, CH)], dsts[s], lin_sems[s])
        pltpu.async_copy(ae_hbm.at[pl.ds(base, CH)], aes[s], lin_sems[s])

    def wait_lin(s):
        pltpu.make_async_copy(src_hbm.at[pl.ds(0, CH)], srcs[s],
                              lin_sems[s]).wait()
        pltpu.make_async_copy(dst_hbm.at[pl.ds(0, CH)], dsts[s],
                              lin_sems[s]).wait()
        pltpu.make_async_copy(ae_hbm.at[pl.ds(0, CH)], aes[s],
                              lin_sems[s]).wait()

    def wait_scatter(b, s):
        pltpu.make_async_copy(rows[b], u_sh.at[dsts[s]], rsc_sems[b]).wait()
        pltpu.make_async_copy(exs[b], den_sh.at[dsts[s]], dsc_sems[b]).wait()

    start_lin(0, 0)

    def quad_body(i, _):
        for sidx in range(4):
            b = sidx & 1
            g = 4 * i + sidx

            @pl.when(g >= 2)
            def _():
                wait_scatter(b, (sidx - 2) % 4)

            wait_lin(sidx)
            pltpu.async_copy(h_hbm.at[srcs[sidx]], rows[b], gat_sems[b])

            @pl.when(g + 1 < NCHUNK)
            def _():
                start_lin(g + 1, (sidx + 1) % 4)

            for k in range(CH // 16):
                s16 = srcs[sidx][pl.ds(k * 16, 16)]
                d16 = dsts[sidx][pl.ds(k * 16, 16)]
                a = (plsc.load_gather(asrc_t, [s16])
                     + plsc.load_gather(adst_t, [d16])
                     + aes[sidx][pl.ds(k * 16, 16)])
                a = jnp.maximum(a, 0.2 * a)
                exs[b][pl.ds(k * 16, 16)] = jnp.exp(a)

            pltpu.make_async_copy(h_hbm.at[srcs[sidx]], rows[b],
                                  gat_sems[b]).wait()

            rb = rows[b]
            eb = exs[b]

            def scale_body(j, _):
                idx16 = jnp.broadcast_to(j, (16,)).astype(jnp.int32)
                sc = plsc.load_gather(eb, [idx16])
                for k in range(D // 16):
                    rb[j, pl.ds(k * 16, 16)] = rb[j, pl.ds(k * 16, 16)] * sc
                return 0

            lax.fori_loop(0, CH, scale_body, 0, unroll=2)
            pltpu.async_copy(rows[b], u_sh.at[dsts[sidx]], rsc_sems[b],
                             add=True)
            pltpu.async_copy(exs[b], den_sh.at[dsts[sidx]], dsc_sems[b],
                             add=True)
        return 0

    lax.fori_loop(0, NCHUNK // 4, quad_body, 0)
    wait_scatter(0, 2)
    wait_scatter(1, 3)

    plsc.subcore_barrier()

    pltpu.sync_copy(u_sh.at[pl.ds(sid * RPT, RPT)],
                    u_out.at[pl.ds(cid * NROWPAD + sid * RPT, RPT)])
    pltpu.sync_copy(den_sh.at[pl.ds(sid * RPT, RPT)],
                    den_out.at[pl.ds(cid * NROWPAD + sid * RPT, RPT)])


@functools.cache
def _sc_edge_kernel():
    return pl.kernel(
        _sc_edge_body,
        out_type=(jax.ShapeDtypeStruct((2 * NROWPAD, D), jnp.float32),
                  jax.ShapeDtypeStruct((2 * NROWPAD,), jnp.float32)),
        mesh=plsc.VectorSubcoreMesh(core_axis_name="c", subcore_axis_name="s",
                                    num_cores=NC, num_subcores=NS),
        compiler_params=pltpu.CompilerParams(needs_layout_passes=False),
        scratch_types=[
            pltpu.VMEM((N,), jnp.float32),
            pltpu.VMEM((N,), jnp.float32),
        ]
        + [pltpu.VMEM((CH,), jnp.int32) for _ in range(4)]
        + [pltpu.VMEM((CH,), jnp.int32) for _ in range(4)]
        + [pltpu.VMEM((CH,), jnp.float32) for _ in range(4)]
        + [pltpu.VMEM((CH,), jnp.float32) for _ in range(2)]
        + [pltpu.VMEM((CH, D), jnp.float32) for _ in range(2)]
        + [
            pltpu.VMEM_SHARED((NROWPAD, D), jnp.float32),
            pltpu.VMEM_SHARED((NROWPAD,), jnp.float32),
        ]
        + [pltpu.SemaphoreType.DMA for _ in range(10)],
    )


def _sc_edge(*args):
    return _sc_edge_kernel()(*args)


def kernel(x, edge_index, edge_attr,
           W1, as1, ad1, We1, ae1, b1,
           W2, as2, ad2, We2, ae2, b2,
           W3, as3, ad3, We3, ae3, b3):
    pad = EPAD - E
    src = jnp.concatenate([edge_index[0], jnp.zeros((pad,), jnp.int32)])
    dst = jnp.concatenate([edge_index[1], jnp.zeros((pad,), jnp.int32)])

    col = lambda v: v.reshape(D, 1)
    aedge3, amean3 = _aedge_call(edge_attr, We1, col(ae1), We2, col(ae2),
                                 We3, col(ae3))
    aepad = jnp.full((3, pad), NEG, jnp.float32)
    aedge3 = jnp.concatenate([aedge3, aepad], axis=1)

    def run_layer(h, av, aedge, amean, b, prev, nxt, is_final):
        asrc = av[:, 0]
        adst = av[:, 1]
        u_all, den_all = _sc_edge(src, dst, aedge, asrc, adst, h)
        u0 = u_all[:N]
        u1 = u_all[NROWPAD:NROWPAD + N]
        d0 = den_all[:N].reshape(N, 1)
        d1 = den_all[NROWPAD:NROWPAD + N].reshape(N, 1)
        return _epi_call(u0, u1, d0, d1, h, av, amean, b.reshape(1, D),
                         prev=prev, nxt=nxt, is_final=is_final)

    h1, av1 = _pre_call(x, W1, col(as1), col(ad1))
    s1, h2, av2 = run_layer(h1, av1, aedge3[0], amean3[0:1, 0:1], b1,
                            None, (W2, col(as2), col(ad2)), False)
    s2, h3, av3 = run_layer(h2, av2, aedge3[1], amean3[1:2, 0:1], b2,
                            s1, (W3, col(as3), col(ad3)), False)
    node_emb, ge = run_layer(h3, av3, aedge3[2], amean3[2:3, 0:1], b3,
                             s2, None, True)
    return node_emb, ge
